# baseline (speedup 1.0000x reference)
import functools

import jax
import jax.numpy as jnp
from jax.experimental import pallas as pl
from jax.experimental.pallas import tpu as pltpu

E = 3200000
D_EDGE = 16
UNITS = 64
ROWS_PER_BLOCK = 25600


def _matmul_block(agg_ref, w_ref, b_ref, out_ref):
    out_ref[...] = (
        jnp.dot(agg_ref[...], w_ref[...], preferred_element_type=jnp.float32)
        + b_ref[...]
    )


def _matmul_bias(agg, w, b):
    grid = (E // ROWS_PER_BLOCK,)
    return pl.pallas_call(
        _matmul_block,
        grid=grid,
        in_specs=[
            pl.BlockSpec((ROWS_PER_BLOCK, D_EDGE), lambda i: (i, 0)),
            pl.BlockSpec((D_EDGE, UNITS), lambda i: (0, 0)),
            pl.BlockSpec((1, UNITS), lambda i: (0, 0)),
        ],
        out_specs=pl.BlockSpec((ROWS_PER_BLOCK, UNITS), lambda i: (i, 0)),
        out_shape=jax.ShapeDtypeStruct((E, UNITS), jnp.float32),
    )(agg, w, b.reshape(1, UNITS))


def kernel(edges_sph_features, edges_neighbor, kernel, bias):
    nbr = edges_neighbor.astype(jnp.int32)
    gathered = jnp.take(edges_sph_features, nbr[:, 1], axis=0)
    agg = jax.ops.segment_sum(gathered, nbr[:, 0], num_segments=E)
    return _matmul_bias(agg, kernel, bias)


# speedup vs baseline: 1.9400x; 1.9400x over previous
import functools

import jax
import jax.numpy as jnp
from jax import lax
from jax.experimental import pallas as pl
from jax.experimental.pallas import tpu as pltpu
from jax.experimental.pallas import tpu_sc as plsc

E = 3200000
D = 16
UNITS = 64

P = 16
W_D = 102400
E_PAD = 2 * P * W_D
TRASH = 512
FK = 512
V = 25
STAGE = FK + V * 16
CH = E // 16
B = 2000
NBLK = CH // B
NSUB = B // (16 * V)
ZROWS = 400
NZCOPY = 16
ROWS_PER_TILE = W_D // 16
NSHIFT = V + 1


def _sc_aggregate(feat, n0, n1):
    mesh = plsc.VectorSubcoreMesh(core_axis_name="c", subcore_axis_name="s")

    @functools.partial(
        pl.kernel,
        mesh=mesh,
        compiler_params=pltpu.CompilerParams(
            use_tc_tiling_on_sc=False, needs_layout_passes=False),
        out_type=jax.ShapeDtypeStruct((E_PAD, D), jnp.float32),
        scratch_types=[
            pltpu.VMEM_SHARED((W_D + TRASH, D), jnp.float32),
            pltpu.VMEM((ZROWS, D), jnp.float32),
            pltpu.VMEM((2, B), jnp.int32),
            pltpu.VMEM((2, B), jnp.int32),
            pltpu.VMEM((STAGE,), jnp.int32),
            pltpu.VMEM((STAGE,), jnp.int32),
            pltpu.VMEM((FK,), jnp.int32),
            pltpu.VMEM((FK,), jnp.int32),
            pltpu.VMEM((FK, D), jnp.float32),
            pltpu.SemaphoreType.DMA,
            pltpu.SemaphoreType.DMA,
            pltpu.SemaphoreType.DMA,
        ],
    )
    def agg_kernel(feat_hbm, n0_hbm, n1_hbm, out_hbm, acc, zbuf, nb0, nb1,
                   s_src, s_dst, g_idx, d_idx, rows, sem_a, sem_b, sem_g):
        c = lax.axis_index("c")
        s = lax.axis_index("s")

        zero16f = jnp.zeros((D,), jnp.float32)

        def zb_body(i, carry):
            zbuf[i, :] = zero16f
            return carry

        lax.fori_loop(0, ZROWS, zb_body, 0)

        iota16 = lax.iota(jnp.int32, 16)
        zero_v = jnp.zeros((16,), jnp.int32)

        def start_blk(blk, par):
            base = s * CH + blk * B

            def s0():
                pltpu.async_copy(n0_hbm.at[pl.ds(base, B)], nb0.at[0], sem_a)
                pltpu.async_copy(n1_hbm.at[pl.ds(base, B)], nb1.at[0], sem_a)

            def s1():
                pltpu.async_copy(n0_hbm.at[pl.ds(base, B)], nb0.at[1], sem_b)
                pltpu.async_copy(n1_hbm.at[pl.ds(base, B)], nb1.at[1], sem_b)

            lax.cond(par == 0, s0, s1)

        def wait_blk(blk, par):
            base = s * CH + blk * B

            def w0():
                pltpu.make_async_copy(n0_hbm.at[pl.ds(base, B)], nb0.at[0],
                                      sem_a).wait()
                pltpu.make_async_copy(n1_hbm.at[pl.ds(base, B)], nb1.at[0],
                                      sem_a).wait()

            def w1():
                pltpu.make_async_copy(n0_hbm.at[pl.ds(base, B)], nb0.at[1],
                                      sem_b).wait()
                pltpu.make_async_copy(n1_hbm.at[pl.ds(base, B)], nb1.at[1],
                                      sem_b).wait()

            lax.cond(par == 0, w0, w1)

        def drain_gather():
            pltpu.make_async_copy(feat_hbm.at[g_idx], rows, sem_g).wait()
            pltpu.sync_copy(rows, acc.at[d_idx], add=True)

        def stage_to_fire_bufs():
            def cp_body(k, carry):
                g_idx[pl.ds(k * 16, 16)] = s_src[pl.ds(k * 16, 16)]
                d_idx[pl.ds(k * 16, 16)] = s_dst[pl.ds(k * 16, 16)]
                return carry

            lax.fori_loop(0, FK // 16, cp_body, 0)

        def pass_body(p, carry):
            wbase = (c * P + p) * W_D
            lo = wbase
            hi = wbase + W_D

            for k in range(NZCOPY):
                pltpu.sync_copy(
                    zbuf, acc.at[pl.ds(s * ROWS_PER_TILE + k * ZROWS,
                                       ZROWS)])
            plsc.subcore_barrier()

            start_blk(0, 0)

            def blk_body(blk, st):
                cur_v, pend = st
                par = jnp.bitwise_and(blk, 1)
                wait_blk(blk, par)
                lax.cond(blk + 1 < NBLK,
                         lambda: start_blk(blk + 1, 1 - par), lambda: None)

                def sub_body(u, st2):
                    cur_v, pend = st2
                    for t in range(V):
                        off = (u * V + t) * 16
                        v0 = nb0[par, pl.ds(off, 16)]
                        v1 = nb1[par, pl.ds(off, 16)]
                        m = (v0 >= lo) & (v0 < hi)
                        mi = m.astype(jnp.int32)
                        excl = plsc.cumsum(mi) - mi
                        pos = cur_v + excl
                        plsc.store_scatter(s_src, [pos], v1, mask=m)
                        plsc.store_scatter(s_dst, [pos], v0 - lo, mask=m)
                        cur_v = cur_v + plsc.all_reduce_population_count(m)
                    cur_sc = jnp.sum(cur_v) >> 4

                    def do_fire(st3):
                        cur_v, pend = st3
                        lax.cond(pend == 1, drain_gather, lambda: None)
                        stage_to_fire_bufs()
                        pltpu.async_copy(feat_hbm.at[g_idx], rows, sem_g)

                        def sh_body(k, carry):
                            t1 = s_src[pl.ds(FK + k * 16, 16)]
                            s_src[pl.ds(k * 16, 16)] = t1
                            t2 = s_dst[pl.ds(FK + k * 16, 16)]
                            s_dst[pl.ds(k * 16, 16)] = t2
                            return carry

                        lax.fori_loop(0, NSHIFT, sh_body, 0)
                        return (cur_v - FK, jnp.int32(1))

                    return lax.cond(cur_sc >= FK, do_fire,
                                    lambda st3: st3, (cur_v, pend))

                return lax.fori_loop(0, NSUB, sub_body, (cur_v, pend))

            cur_v, pend = lax.fori_loop(0, NBLK, blk_body,
                                        (zero_v, jnp.int32(0)))

            cur_sc = jnp.sum(cur_v) >> 4

            def pad_body(k, carry):
                j16 = k * 16 + iota16
                keep = j16 < cur_sc
                vsrc = s_src[pl.ds(k * 16, 16)]
                s_src[pl.ds(k * 16, 16)] = jnp.where(keep, vsrc, s * FK + j16)
                vdst = s_dst[pl.ds(k * 16, 16)]
                s_dst[pl.ds(k * 16, 16)] = jnp.where(keep, vdst, W_D + j16)
                return carry

            lax.fori_loop(0, FK // 16, pad_body, 0)
            lax.cond(pend == 1, drain_gather, lambda: None)
            stage_to_fire_bufs()
            pltpu.async_copy(feat_hbm.at[g_idx], rows, sem_g).wait()
            pltpu.sync_copy(rows, acc.at[d_idx], add=True)
            plsc.subcore_barrier()

            pltpu.sync_copy(
                acc.at[pl.ds(s * ROWS_PER_TILE, ROWS_PER_TILE)],
                out_hbm.at[pl.ds(wbase + s * ROWS_PER_TILE, ROWS_PER_TILE)])
            return carry

        lax.fori_loop(0, P, pass_body, 0)

    return agg_kernel(feat, n0, n1)


ROWS_PER_BLOCK = 25600


def _matmul_block(agg_ref, w_ref, b_ref, out_ref):
    out_ref[...] = (
        jnp.dot(agg_ref[...], w_ref[...], preferred_element_type=jnp.float32)
        + b_ref[...]
    )


def _matmul_bias(agg, w, b):
    grid = (E // ROWS_PER_BLOCK,)
    return pl.pallas_call(
        _matmul_block,
        grid=grid,
        in_specs=[
            pl.BlockSpec((ROWS_PER_BLOCK, D), lambda i: (i, 0)),
            pl.BlockSpec((D, UNITS), lambda i: (0, 0)),
            pl.BlockSpec((1, UNITS), lambda i: (0, 0)),
        ],
        out_specs=pl.BlockSpec((ROWS_PER_BLOCK, UNITS), lambda i: (i, 0)),
        out_shape=jax.ShapeDtypeStruct((E, UNITS), jnp.float32),
    )(agg, w, b.reshape(1, UNITS))


def kernel(edges_sph_features, edges_neighbor, kernel, bias):
    nbr = edges_neighbor.astype(jnp.int32)
    n0 = nbr[:, 0]
    n1 = nbr[:, 1]
    agg = _sc_aggregate(edges_sph_features, n0, n1)
    return _matmul_bias(agg, kernel, bias)


# speedup vs baseline: 2.3982x; 1.2362x over previous
import functools

import jax
import jax.numpy as jnp
from jax import lax
from jax.experimental import pallas as pl
from jax.experimental.pallas import tpu as pltpu
from jax.experimental.pallas import tpu_sc as plsc

E = 3200000
D = 16
UNITS = 64

P = 18
W_D = 89600
E_PAD = 2 * P * W_D
TRASH = 1024
CAP = 64
NFIRE = CAP * 16
V = 25
FIRE_POS = (CAP - V) * 16
CH = E // 16
B = 2000
NBLK = CH // B
NSUB = B // (16 * V)
ZROWS = 560
NZCOPY = 10
ROWS_PER_TILE = W_D // 16


def _sc_aggregate(feat, n0, n1):
    mesh = plsc.VectorSubcoreMesh(core_axis_name="c", subcore_axis_name="s")

    @functools.partial(
        pl.kernel,
        mesh=mesh,
        compiler_params=pltpu.CompilerParams(
            use_tc_tiling_on_sc=False, needs_layout_passes=False),
        out_type=jax.ShapeDtypeStruct((E_PAD, D), jnp.float32),
        scratch_types=[
            pltpu.VMEM_SHARED((W_D + TRASH, D), jnp.float32),
            pltpu.VMEM((ZROWS, D), jnp.float32),
            pltpu.VMEM((2, B), jnp.int32),
            pltpu.VMEM((2, B), jnp.int32),
            pltpu.VMEM((NFIRE,), jnp.int32),
            pltpu.VMEM((NFIRE,), jnp.int32),
            pltpu.VMEM((NFIRE,), jnp.int32),
            pltpu.VMEM((NFIRE,), jnp.int32),
            pltpu.VMEM((NFIRE, D), jnp.float32),
            pltpu.SemaphoreType.DMA,
            pltpu.SemaphoreType.DMA,
            pltpu.SemaphoreType.DMA,
        ],
    )
    def agg_kernel(feat_hbm, n0_hbm, n1_hbm, out_hbm, acc, zbuf, nb0, nb1,
                   s_src, s_dst, g_idx, d_idx, rows, sem_a, sem_b, sem_g):
        c = lax.axis_index("c")
        s = lax.axis_index("s")

        zero16f = jnp.zeros((D,), jnp.float32)

        def zb_body(i, carry):
            zbuf[i, :] = zero16f
            return carry

        lax.fori_loop(0, ZROWS, zb_body, 0)

        iota16 = lax.iota(jnp.int32, 16)
        pad_src0 = s * NFIRE + iota16

        def start_blk(blk, par):
            base = s * CH + blk * B

            def s0():
                pltpu.async_copy(n0_hbm.at[pl.ds(base, B)], nb0.at[0], sem_a)
                pltpu.async_copy(n1_hbm.at[pl.ds(base, B)], nb1.at[0], sem_a)

            def s1():
                pltpu.async_copy(n0_hbm.at[pl.ds(base, B)], nb0.at[1], sem_b)
                pltpu.async_copy(n1_hbm.at[pl.ds(base, B)], nb1.at[1], sem_b)

            lax.cond(par == 0, s0, s1)

        def wait_blk(blk, par):
            base = s * CH + blk * B

            def w0():
                pltpu.make_async_copy(n0_hbm.at[pl.ds(base, B)], nb0.at[0],
                                      sem_a).wait()
                pltpu.make_async_copy(n1_hbm.at[pl.ds(base, B)], nb1.at[0],
                                      sem_a).wait()

            def w1():
                pltpu.make_async_copy(n0_hbm.at[pl.ds(base, B)], nb0.at[1],
                                      sem_b).wait()
                pltpu.make_async_copy(n1_hbm.at[pl.ds(base, B)], nb1.at[1],
                                      sem_b).wait()

            lax.cond(par == 0, w0, w1)

        def drain_gather():
            pltpu.make_async_copy(feat_hbm.at[g_idx], rows, sem_g).wait()
            pltpu.sync_copy(rows, acc.at[d_idx], add=True)

        def stage_copy_and_prefill():
            def cp_body(k, carry):
                k16 = k * 16
                g_idx[pl.ds(k16, 16)] = s_src[pl.ds(k16, 16)]
                d_idx[pl.ds(k16, 16)] = s_dst[pl.ds(k16, 16)]
                s_src[pl.ds(k16, 16)] = pad_src0 + k16
                s_dst[pl.ds(k16, 16)] = (W_D + k16) + iota16
                return carry

            lax.fori_loop(0, NFIRE // 16, cp_body, 0)

        def prefill_stage():
            def pf_body(k, carry):
                k16 = k * 16
                s_src[pl.ds(k16, 16)] = pad_src0 + k16
                s_dst[pl.ds(k16, 16)] = (W_D + k16) + iota16
                return carry

            lax.fori_loop(0, NFIRE // 16, pf_body, 0)

        def pass_body(p, carry):
            wbase = (c * P + p) * W_D
            lo = wbase
            hi = wbase + W_D

            for k in range(NZCOPY):
                pltpu.sync_copy(
                    zbuf, acc.at[pl.ds(s * ROWS_PER_TILE + k * ZROWS,
                                       ZROWS)])
            plsc.subcore_barrier()
            prefill_stage()

            start_blk(0, 0)

            def blk_body(blk, st):
                pos_v, pend = st
                par = jnp.bitwise_and(blk, 1)
                wait_blk(blk, par)
                lax.cond(blk + 1 < NBLK,
                         lambda: start_blk(blk + 1, 1 - par), lambda: None)

                def sub_body(u, st2):
                    pos_v, pend = st2
                    for t in range(V):
                        off = (u * V + t) * 16
                        v0 = nb0[par, pl.ds(off, 16)]
                        v1 = nb1[par, pl.ds(off, 16)]
                        m = (v0 >= lo) & (v0 < hi)
                        plsc.store_scatter(s_src, [pos_v], v1, mask=m)
                        plsc.store_scatter(s_dst, [pos_v], v0 - lo, mask=m)
                        pos_v = pos_v + jnp.where(m, 16, 0)
                    mx = jnp.max(pos_v)

                    def do_fire(st3):
                        pos_v, pend = st3
                        lax.cond(pend == 1, drain_gather, lambda: None)
                        stage_copy_and_prefill()
                        pltpu.async_copy(feat_hbm.at[g_idx], rows, sem_g)
                        return (iota16, jnp.int32(1))

                    return lax.cond(mx >= FIRE_POS, do_fire,
                                    lambda st3: st3, (pos_v, pend))

                return lax.fori_loop(0, NSUB, sub_body, st)

            pos_v, pend = lax.fori_loop(0, NBLK, blk_body,
                                        (iota16, jnp.int32(0)))

            lax.cond(pend == 1, drain_gather, lambda: None)
            stage_copy_and_prefill()
            pltpu.async_copy(feat_hbm.at[g_idx], rows, sem_g).wait()
            pltpu.sync_copy(rows, acc.at[d_idx], add=True)
            plsc.subcore_barrier()

            pltpu.sync_copy(
                acc.at[pl.ds(s * ROWS_PER_TILE, ROWS_PER_TILE)],
                out_hbm.at[pl.ds(wbase + s * ROWS_PER_TILE, ROWS_PER_TILE)])
            return carry

        lax.fori_loop(0, P, pass_body, 0)

    return agg_kernel(feat, n0, n1)


RB8 = 3200
NBLOCKS = (E // 8) // RB8


def _matmul_block(agg_ref, w_ref, b_ref, out_ref):
    out_ref[...] = (
        jnp.dot(agg_ref[...], w_ref[...], preferred_element_type=jnp.float32)
        + b_ref[...]
    )


def _matmul_bias(agg128, w128, b512):
    return pl.pallas_call(
        _matmul_block,
        grid=(NBLOCKS,),
        in_specs=[
            pl.BlockSpec((RB8, 128), lambda i: (i, 0)),
            pl.BlockSpec((128, 8 * UNITS), lambda i: (0, 0)),
            pl.BlockSpec((1, 8 * UNITS), lambda i: (0, 0)),
        ],
        out_specs=pl.BlockSpec((RB8, 8 * UNITS), lambda i: (i, 0)),
        out_shape=jax.ShapeDtypeStruct((E // 8, 8 * UNITS), jnp.float32),
    )(agg128, w128, b512)


def kernel(edges_sph_features, edges_neighbor, kernel, bias):
    nbr = edges_neighbor.astype(jnp.int32)
    n0 = nbr[:, 0]
    n1 = nbr[:, 1]
    agg = _sc_aggregate(edges_sph_features, n0, n1)
    agg128 = jnp.reshape(agg, (E_PAD // 8, 8 * D))
    w128 = jnp.kron(jnp.eye(8, dtype=jnp.float32), kernel)
    b512 = jnp.tile(bias, 8).reshape(1, 8 * UNITS)
    out = _matmul_bias(agg128, w128, b512)
    return jnp.reshape(out, (E, UNITS))


# speedup vs baseline: 2.6904x; 1.1219x over previous
import functools

import jax
import jax.numpy as jnp
from jax import lax
from jax.experimental import pallas as pl
from jax.experimental.pallas import tpu as pltpu
from jax.experimental.pallas import tpu_sc as plsc

E = 3200000
D = 16
UNITS = 64

P = 18
W_D = 89600
E_PAD = 2 * P * W_D
TRASH = 1024
CAP = 64
NFIRE = CAP * 16
V = 25
FIRE_POS = (CAP - V) * 16
CH = E // 16
B = 2000
NBLK = CH // B
NSUB = B // (16 * V)
ZROWS = 560
NZCOPY = 10
ROWS_PER_TILE = W_D // 16


def _sc_aggregate(feat, n0, n1):
    mesh = plsc.VectorSubcoreMesh(core_axis_name="c", subcore_axis_name="s")

    @functools.partial(
        pl.kernel,
        mesh=mesh,
        compiler_params=pltpu.CompilerParams(
            use_tc_tiling_on_sc=False, needs_layout_passes=False),
        out_type=jax.ShapeDtypeStruct((E_PAD, D), jnp.float32),
        scratch_types=[
            pltpu.VMEM_SHARED((W_D + TRASH, D), jnp.float32),
            pltpu.VMEM((ZROWS, D), jnp.float32),
            pltpu.VMEM((2, B), jnp.int32),
            pltpu.VMEM((2, B), jnp.int32),
            pltpu.VMEM((NFIRE,), jnp.int32),
            pltpu.VMEM((NFIRE,), jnp.int32),
            pltpu.VMEM((NFIRE,), jnp.int32),
            pltpu.VMEM((NFIRE,), jnp.int32),
            pltpu.VMEM((NFIRE, D), jnp.float32),
            pltpu.SemaphoreType.DMA,
            pltpu.SemaphoreType.DMA,
            pltpu.SemaphoreType.DMA,
        ],
    )
    def agg_kernel(feat_hbm, n0_hbm, n1_hbm, out_hbm, acc, zbuf, nb0, nb1,
                   s_src, s_dst, g_idx, d_idx, rows, sem_a, sem_b, sem_g):
        c = lax.axis_index("c")
        s = lax.axis_index("s")

        zero16f = jnp.zeros((D,), jnp.float32)

        def zb_body(i, carry):
            zbuf[i, :] = zero16f
            return carry

        lax.fori_loop(0, ZROWS, zb_body, 0)

        iota16 = lax.iota(jnp.int32, 16)
        pad_src0 = s * NFIRE + iota16

        def start_blk(blk, par):
            base = s * CH + blk * B

            def s0():
                pltpu.async_copy(n0_hbm.at[pl.ds(base, B)], nb0.at[0], sem_a)
                pltpu.async_copy(n1_hbm.at[pl.ds(base, B)], nb1.at[0], sem_a)

            def s1():
                pltpu.async_copy(n0_hbm.at[pl.ds(base, B)], nb0.at[1], sem_b)
                pltpu.async_copy(n1_hbm.at[pl.ds(base, B)], nb1.at[1], sem_b)

            lax.cond(par == 0, s0, s1)

        def wait_blk(blk, par):
            base = s * CH + blk * B

            def w0():
                pltpu.make_async_copy(n0_hbm.at[pl.ds(base, B)], nb0.at[0],
                                      sem_a).wait()
                pltpu.make_async_copy(n1_hbm.at[pl.ds(base, B)], nb1.at[0],
                                      sem_a).wait()

            def w1():
                pltpu.make_async_copy(n0_hbm.at[pl.ds(base, B)], nb0.at[1],
                                      sem_b).wait()
                pltpu.make_async_copy(n1_hbm.at[pl.ds(base, B)], nb1.at[1],
                                      sem_b).wait()

            lax.cond(par == 0, w0, w1)

        def drain_gather():
            pltpu.make_async_copy(feat_hbm.at[g_idx], rows, sem_g).wait()
            pltpu.sync_copy(rows, acc.at[d_idx], add=True)

        def stage_copy_and_prefill():
            def cp_body(k, carry):
                k16 = k * 16
                g_idx[pl.ds(k16, 16)] = s_src[pl.ds(k16, 16)]
                d_idx[pl.ds(k16, 16)] = s_dst[pl.ds(k16, 16)]
                s_src[pl.ds(k16, 16)] = pad_src0 + k16
                s_dst[pl.ds(k16, 16)] = (W_D + k16) + iota16
                return carry

            lax.fori_loop(0, NFIRE // 16, cp_body, 0)

        def prefill_stage():
            def pf_body(k, carry):
                k16 = k * 16
                s_src[pl.ds(k16, 16)] = pad_src0 + k16
                s_dst[pl.ds(k16, 16)] = (W_D + k16) + iota16
                return carry

            lax.fori_loop(0, NFIRE // 16, pf_body, 0)

        def pass_body(p, carry):
            wbase = (c * P + p) * W_D
            lo = wbase
            hi = wbase + W_D

            for k in range(NZCOPY):
                pltpu.sync_copy(
                    zbuf, acc.at[pl.ds(s * ROWS_PER_TILE + k * ZROWS,
                                       ZROWS)])
            plsc.subcore_barrier()
            prefill_stage()

            start_blk(0, 0)

            def blk_body(blk, st):
                pos_v, pend = st
                par = jnp.bitwise_and(blk, 1)
                wait_blk(blk, par)
                lax.cond(blk + 1 < NBLK,
                         lambda: start_blk(blk + 1, 1 - par), lambda: None)

                def sub_body(u, st2):
                    pos_v, pend = st2
                    base_off = u * (V * 16)
                    nxt0 = nb0[par, pl.ds(base_off, 16)]
                    nxt1 = nb1[par, pl.ds(base_off, 16)]
                    for t in range(V):
                        v0 = nxt0
                        v1 = nxt1
                        if t + 1 < V:
                            off2 = base_off + (t + 1) * 16
                            nxt0 = nb0[par, pl.ds(off2, 16)]
                            nxt1 = nb1[par, pl.ds(off2, 16)]
                        m = (v0 >= lo) & (v0 < hi)
                        plsc.store_scatter(s_src, [pos_v], v1, mask=m)
                        plsc.store_scatter(s_dst, [pos_v], v0 - lo, mask=m)
                        pos_v = pos_v + jnp.where(m, 16, 0)
                    mx = jnp.max(pos_v)

                    def do_fire(st3):
                        pos_v, pend = st3
                        lax.cond(pend == 1, drain_gather, lambda: None)
                        stage_copy_and_prefill()
                        pltpu.async_copy(feat_hbm.at[g_idx], rows, sem_g)
                        return (iota16, jnp.int32(1))

                    return lax.cond(mx >= FIRE_POS, do_fire,
                                    lambda st3: st3, (pos_v, pend))

                return lax.fori_loop(0, NSUB, sub_body, st)

            pos_v, pend = lax.fori_loop(0, NBLK, blk_body,
                                        (iota16, jnp.int32(0)))

            lax.cond(pend == 1, drain_gather, lambda: None)
            stage_copy_and_prefill()
            pltpu.async_copy(feat_hbm.at[g_idx], rows, sem_g).wait()
            pltpu.sync_copy(rows, acc.at[d_idx], add=True)
            plsc.subcore_barrier()

            pltpu.sync_copy(
                acc.at[pl.ds(s * ROWS_PER_TILE, ROWS_PER_TILE)],
                out_hbm.at[pl.ds(wbase + s * ROWS_PER_TILE, ROWS_PER_TILE)])
            return carry

        lax.fori_loop(0, P, pass_body, 0)

    return agg_kernel(feat, n0, n1)


RB8 = 3200
NBLOCKS = (E // 8) // RB8


def _matmul_block(agg_ref, w_ref, b_ref, out_ref):
    a = agg_ref[...].reshape(RB8, 128)
    out_ref[...] = (
        jnp.dot(a, w_ref[...], preferred_element_type=jnp.float32)
        + b_ref[...]
    )


def _matmul_bias(agg1d, w128, b512):
    return pl.pallas_call(
        _matmul_block,
        grid=(NBLOCKS,),
        in_specs=[
            pl.BlockSpec((RB8 * 128,), lambda i: (i,)),
            pl.BlockSpec((128, 8 * UNITS), lambda i: (0, 0)),
            pl.BlockSpec((1, 8 * UNITS), lambda i: (0, 0)),
        ],
        out_specs=pl.BlockSpec((RB8, 8 * UNITS), lambda i: (i, 0)),
        out_shape=jax.ShapeDtypeStruct((E // 8, 8 * UNITS), jnp.float32),
    )(agg1d, w128, b512)


def kernel(edges_sph_features, edges_neighbor, kernel, bias):
    nbr = edges_neighbor.astype(jnp.int32)
    n0 = nbr[:, 0]
    n1 = nbr[:, 1]
    agg = _sc_aggregate(edges_sph_features, n0, n1)
    agg1d = jnp.reshape(agg, (E_PAD * D,))
    w128 = jnp.kron(jnp.eye(8, dtype=jnp.float32), kernel)
    b512 = jnp.tile(bias, 8).reshape(1, 8 * UNITS)
    out = _matmul_bias(agg1d, w128, b512)
    return jnp.reshape(out, (E, UNITS))


# speedup vs baseline: 2.6951x; 1.0017x over previous
import functools

import jax
import jax.numpy as jnp
from jax import lax
from jax.experimental import pallas as pl
from jax.experimental.pallas import tpu as pltpu
from jax.experimental.pallas import tpu_sc as plsc

E = 3200000
D = 16
UNITS = 64

P = 18
W_D = 89600
E_PAD = 2 * P * W_D
TRASH = 1024
CAP = 64
NFIRE = CAP * 16
V = 25
FIRE_POS = (CAP - V) * 16
CH = E // 16
B = 2000
NBLK = CH // B
NSUB = B // (16 * V)
ZROWS = 560
NZCOPY = 10
ROWS_PER_TILE = W_D // 16


def _sc_aggregate(feat, n0, n1):
    mesh = plsc.VectorSubcoreMesh(core_axis_name="c", subcore_axis_name="s")

    @functools.partial(
        pl.kernel,
        mesh=mesh,
        compiler_params=pltpu.CompilerParams(
            use_tc_tiling_on_sc=False, needs_layout_passes=False),
        out_type=jax.ShapeDtypeStruct((E_PAD, D), jnp.float32),
        scratch_types=[
            pltpu.VMEM_SHARED((W_D + TRASH, D), jnp.float32),
            pltpu.VMEM((ZROWS, D), jnp.float32),
            pltpu.VMEM((2, B), jnp.int32),
            pltpu.VMEM((2, B), jnp.int32),
            pltpu.VMEM((NFIRE,), jnp.int32),
            pltpu.VMEM((NFIRE,), jnp.int32),
            pltpu.VMEM((NFIRE,), jnp.int32),
            pltpu.VMEM((NFIRE,), jnp.int32),
            pltpu.VMEM((NFIRE, D), jnp.float32),
            pltpu.SemaphoreType.DMA,
            pltpu.SemaphoreType.DMA,
            pltpu.SemaphoreType.DMA,
        ],
    )
    def agg_kernel(feat_hbm, n0_hbm, n1_hbm, out_hbm, acc, zbuf, nb0, nb1,
                   s_src, s_dst, g_idx, d_idx, rows, sem_a, sem_b, sem_g):
        c = lax.axis_index("c")
        s = lax.axis_index("s")

        zero16f = jnp.zeros((D,), jnp.float32)

        def zb_body(i, carry):
            zbuf[i, :] = zero16f
            return carry

        lax.fori_loop(0, ZROWS, zb_body, 0)

        iota16 = lax.iota(jnp.int32, 16)
        pad_src0 = s * NFIRE + iota16

        def start_blk(blk, par):
            base = s * CH + blk * B

            def s0():
                pltpu.async_copy(n0_hbm.at[pl.ds(base, B)], nb0.at[0], sem_a)
                pltpu.async_copy(n1_hbm.at[pl.ds(base, B)], nb1.at[0], sem_a)

            def s1():
                pltpu.async_copy(n0_hbm.at[pl.ds(base, B)], nb0.at[1], sem_b)
                pltpu.async_copy(n1_hbm.at[pl.ds(base, B)], nb1.at[1], sem_b)

            lax.cond(par == 0, s0, s1)

        def wait_blk(blk, par):
            base = s * CH + blk * B

            def w0():
                pltpu.make_async_copy(n0_hbm.at[pl.ds(base, B)], nb0.at[0],
                                      sem_a).wait()
                pltpu.make_async_copy(n1_hbm.at[pl.ds(base, B)], nb1.at[0],
                                      sem_a).wait()

            def w1():
                pltpu.make_async_copy(n0_hbm.at[pl.ds(base, B)], nb0.at[1],
                                      sem_b).wait()
                pltpu.make_async_copy(n1_hbm.at[pl.ds(base, B)], nb1.at[1],
                                      sem_b).wait()

            lax.cond(par == 0, w0, w1)

        def drain_gather():
            pltpu.make_async_copy(feat_hbm.at[g_idx], rows, sem_g).wait()
            pltpu.sync_copy(rows, acc.at[d_idx], add=True)

        def stage_copy_and_prefill():
            def cp_body(k, carry):
                k16 = k * 16
                g_idx[pl.ds(k16, 16)] = s_src[pl.ds(k16, 16)]
                d_idx[pl.ds(k16, 16)] = s_dst[pl.ds(k16, 16)]
                s_src[pl.ds(k16, 16)] = pad_src0 + k16
                s_dst[pl.ds(k16, 16)] = (W_D + k16) + iota16
                return carry

            lax.fori_loop(0, NFIRE // 16, cp_body, 0)

        def prefill_stage():
            def pf_body(k, carry):
                k16 = k * 16
                s_src[pl.ds(k16, 16)] = pad_src0 + k16
                s_dst[pl.ds(k16, 16)] = (W_D + k16) + iota16
                return carry

            lax.fori_loop(0, NFIRE // 16, pf_body, 0)

        def pass_body(p, carry):
            wbase = (c * P + p) * W_D
            lo = wbase
            hi = wbase + W_D

            for k in range(NZCOPY):
                pltpu.async_copy(
                    zbuf, acc.at[pl.ds(s * ROWS_PER_TILE + k * ZROWS,
                                       ZROWS)], sem_g)
            for k in range(NZCOPY):
                pltpu.make_async_copy(
                    zbuf, acc.at[pl.ds(s * ROWS_PER_TILE + k * ZROWS,
                                       ZROWS)], sem_g).wait()
            plsc.subcore_barrier()
            prefill_stage()

            start_blk(0, 0)

            def blk_body(blk, st):
                pos_v, pend = st
                par = jnp.bitwise_and(blk, 1)
                wait_blk(blk, par)
                lax.cond(blk + 1 < NBLK,
                         lambda: start_blk(blk + 1, 1 - par), lambda: None)

                def sub_body(u, st2):
                    pos_v, pend = st2
                    base_off = u * (V * 16)
                    nxt0 = nb0[par, pl.ds(base_off, 16)]
                    nxt1 = nb1[par, pl.ds(base_off, 16)]
                    for t in range(V):
                        v0 = nxt0
                        v1 = nxt1
                        if t + 1 < V:
                            off2 = base_off + (t + 1) * 16
                            nxt0 = nb0[par, pl.ds(off2, 16)]
                            nxt1 = nb1[par, pl.ds(off2, 16)]
                        m = (v0 >= lo) & (v0 < hi)
                        plsc.store_scatter(s_src, [pos_v], v1, mask=m)
                        plsc.store_scatter(s_dst, [pos_v], v0 - lo, mask=m)
                        pos_v = pos_v + jnp.where(m, 16, 0)
                    mx = jnp.max(pos_v)

                    def do_fire(st3):
                        pos_v, pend = st3
                        lax.cond(pend == 1, drain_gather, lambda: None)
                        stage_copy_and_prefill()
                        pltpu.async_copy(feat_hbm.at[g_idx], rows, sem_g)
                        return (iota16, jnp.int32(1))

                    return lax.cond(mx >= FIRE_POS, do_fire,
                                    lambda st3: st3, (pos_v, pend))

                return lax.fori_loop(0, NSUB, sub_body, st)

            pos_v, pend = lax.fori_loop(0, NBLK, blk_body,
                                        (iota16, jnp.int32(0)))

            lax.cond(pend == 1, drain_gather, lambda: None)
            stage_copy_and_prefill()
            pltpu.async_copy(feat_hbm.at[g_idx], rows, sem_g).wait()
            pltpu.sync_copy(rows, acc.at[d_idx], add=True)
            plsc.subcore_barrier()

            pltpu.sync_copy(
                acc.at[pl.ds(s * ROWS_PER_TILE, ROWS_PER_TILE)],
                out_hbm.at[pl.ds(wbase + s * ROWS_PER_TILE, ROWS_PER_TILE)])
            return carry

        lax.fori_loop(0, P, pass_body, 0)

    return agg_kernel(feat, n0, n1)


RB8 = 3200
NBLOCKS = (E // 8) // RB8


def _matmul_block(agg_ref, w_ref, b_ref, out_ref):
    a = agg_ref[...].reshape(RB8, 128)
    out_ref[...] = (
        jnp.dot(a, w_ref[...], preferred_element_type=jnp.float32)
        + b_ref[...]
    )


def _matmul_bias(agg1d, w128, b512):
    return pl.pallas_call(
        _matmul_block,
        grid=(NBLOCKS,),
        in_specs=[
            pl.BlockSpec((RB8 * 128,), lambda i: (i,)),
            pl.BlockSpec((128, 8 * UNITS), lambda i: (0, 0)),
            pl.BlockSpec((1, 8 * UNITS), lambda i: (0, 0)),
        ],
        out_specs=pl.BlockSpec((RB8, 8 * UNITS), lambda i: (i, 0)),
        out_shape=jax.ShapeDtypeStruct((E // 8, 8 * UNITS), jnp.float32),
    )(agg1d, w128, b512)


def kernel(edges_sph_features, edges_neighbor, kernel, bias):
    nbr = edges_neighbor.astype(jnp.int32)
    n0 = nbr[:, 0]
    n1 = nbr[:, 1]
    agg = _sc_aggregate(edges_sph_features, n0, n1)
    agg1d = jnp.reshape(agg, (E_PAD * D,))
    w128 = jnp.kron(jnp.eye(8, dtype=jnp.float32), kernel)
    b512 = jnp.tile(bias, 8).reshape(1, 8 * UNITS)
    out = _matmul_bias(agg1d, w128, b512)
    return jnp.reshape(out, (E, UNITS))
